# baseline (speedup 1.0000x reference)
import functools
import math

import jax
import jax.numpy as jnp
from jax import lax
from jax.experimental import pallas as pl
from jax.experimental.pallas import tpu as pltpu
from jax.experimental.pallas import tpu_sc as plsc

N = 10000
E = 320000
D = 128
L = 16
CHUNK = 128
NC = 2
NS = 16
NW = NC * NS
NUM_CHUNKS = E // CHUNK
CPW = -(-NUM_CHUNKS // NW)
CPW = CPW + (CPW % 2)
E_PAD = NW * CPW * CHUNK
ROWCH = 200
NRC = N // ROWCH
RC_PER_SUB = -(-NRC // NS)
SM_ROWS = E_PAD * L // 128
CA = 64
HPW = CPW * 2


def _qkv_body(x_ref, w_ref, b_ref, q_ref, k_ref, v_ref):
    y = jnp.dot(x_ref[...], w_ref[...], preferred_element_type=jnp.float32)
    y = y + b_ref[...]
    q_ref[...] = y[:, 0:D]
    k_ref[...] = y[:, D:2 * D]
    v_ref[...] = y[:, 2 * D:3 * D]


def _qkv_call(x, wcat, bcat):
    blk = 1000
    return pl.pallas_call(
        _qkv_body,
        grid=(N // blk,),
        in_specs=[
            pl.BlockSpec((blk, D), lambda i: (i, 0)),
            pl.BlockSpec((D, 3 * D), lambda i: (0, 0)),
            pl.BlockSpec((1, 3 * D), lambda i: (0, 0)),
        ],
        out_specs=[pl.BlockSpec((blk, D), lambda i: (i, 0))] * 3,
        out_shape=[jax.ShapeDtypeStruct((N, D), jnp.float32)] * 3,
    )(x, wcat, bcat)


def _elu_qkv_body(p0_ref, p1_ref, w_ref, b_ref, q_ref, k_ref, v_ref):
    h = p0_ref[...] + p1_ref[...]
    h = jnp.where(h > 0, h, jnp.exp(jnp.minimum(h, 0.0)) - 1.0)
    y = jnp.dot(h, w_ref[...], preferred_element_type=jnp.float32)
    y = y + b_ref[...]
    q_ref[...] = y[:, 0:D]
    k_ref[...] = y[:, D:2 * D]
    v_ref[...] = y[:, 2 * D:3 * D]


def _elu_qkv_call(p0, p1, wcat, bcat):
    blk = 1000
    return pl.pallas_call(
        _elu_qkv_body,
        grid=(N // blk,),
        in_specs=[
            pl.BlockSpec((blk, D), lambda i: (i, 0)),
            pl.BlockSpec((blk, D), lambda i: (i, 0)),
            pl.BlockSpec((D, 3 * D), lambda i: (0, 0)),
            pl.BlockSpec((1, 3 * D), lambda i: (0, 0)),
        ],
        out_specs=[pl.BlockSpec((blk, D), lambda i: (i, 0))] * 3,
        out_shape=[jax.ShapeDtypeStruct((N, D), jnp.float32)] * 3,
    )(p0, p1, wcat, bcat)


def _softmax_body(p_ref, rmat_ref, a_ref):
    s = jnp.dot(p_ref[...], rmat_ref[...], preferred_element_type=jnp.float32)
    m = jnp.max(s)
    ex = jnp.exp(s - m)
    a_ref[...] = ex * (1.0 / jnp.sum(ex))


def _softmax_call(p16, rmat):
    p2 = p16.reshape(NW * CPW, CHUNK * L)
    return pl.pallas_call(
        _softmax_body,
        in_specs=[
            pl.BlockSpec((NW * CPW, CHUNK * L), lambda: (0, 0)),
            pl.BlockSpec((CHUNK * L, CHUNK), lambda: (0, 0)),
        ],
        out_specs=pl.BlockSpec((NW * CPW, CHUNK), lambda: (0, 0)),
        out_shape=jax.ShapeDtypeStruct((NW * CPW, CHUNK), jnp.float32),
    )(p2, rmat)


def _add_body(p0_ref, p1_ref, o_ref):
    o_ref[...] = p0_ref[...] + p1_ref[...]


def _add_call(p0, p1):
    blk = 1000
    return pl.pallas_call(
        _add_body,
        grid=(N // blk,),
        in_specs=[pl.BlockSpec((blk, D), lambda i: (i, 0))] * 2,
        out_specs=pl.BlockSpec((blk, D), lambda i: (i, 0)),
        out_shape=jax.ShapeDtypeStruct((N, D), jnp.float32),
    )(p0, p1)


_MESH = plsc.VectorSubcoreMesh(core_axis_name="c", subcore_axis_name="s")


def _scores_body(q_hbm, k_hbm, row_hbm, col_hbm, p16_hbm,
                 idxr, idxc, qr, kc, sout, isem, gsem, wsem):
    core = lax.axis_index("c")
    sub = lax.axis_index("s")
    wid = core * NS + sub

    def valid(i):
        return i * NW + wid < NUM_CHUNKS

    def issue_idx(slot, i):
        @pl.when(i < CPW)
        def _():
            pltpu.async_copy(row_hbm.at[wid * CPW + i], idxr.at[slot], isem)
            pltpu.async_copy(col_hbm.at[wid * CPW + i], idxc.at[slot], isem)

    def wait_idx(slot, i):
        @pl.when(i < CPW)
        def _():
            pltpu.make_async_copy(row_hbm.at[wid * CPW + i], idxr.at[slot], isem).wait()
            pltpu.make_async_copy(col_hbm.at[wid * CPW + i], idxc.at[slot], isem).wait()

    def issue(slot, i):
        @pl.when(valid(i))
        def _():
            pltpu.async_copy(q_hbm.at[idxr.at[slot]], qr.at[slot], gsem)
            pltpu.async_copy(k_hbm.at[idxc.at[slot]], kc.at[slot], gsem)

    def wait_gathers(slot, i):
        @pl.when(valid(i))
        def _():
            pltpu.make_async_copy(q_hbm.at[idxr.at[slot]], qr.at[slot], gsem).wait()
            pltpu.make_async_copy(k_hbm.at[idxc.at[slot]], kc.at[slot], gsem).wait()

    pltpu.sync_copy(row_hbm.at[wid * CPW], idxr.at[0])
    pltpu.sync_copy(col_hbm.at[wid * CPW], idxc.at[0])
    issue(0, 0)
    issue_idx(1, 1)

    def pair_body(p, carry):
        for b in range(2):
            i = p * 2 + b
            wait_gathers(b, i)
            wait_idx(1 - b, i + 1)
            issue(1 - b, i + 1)
            issue_idx(b, i + 2)

            @pl.when(i >= 2)
            def _():
                pltpu.make_async_copy(
                    sout.at[b],
                    p16_hbm.at[pl.ds((wid * CPW + i - 2) * CHUNK, CHUNK)],
                    wsem).wait()

            @pl.when(valid(i))
            def _():
                @plsc.parallel_loop(0, CHUNK // L, unroll=2)
                def _compute(grp):
                    for eo in range(L):
                        e = grp * L + eo
                        acc = qr[b, e, pl.ds(0, L)] * kc[b, e, pl.ds(0, L)]
                        for d in range(1, D // L):
                            acc = acc + (qr[b, e, pl.ds(d * L, L)] *
                                         kc[b, e, pl.ds(d * L, L)])
                        sout[b, e, :] = acc

            @pl.when(jnp.logical_not(valid(i)))
            def _():
                neg = jnp.full((L,), -1.0e30, jnp.float32)

                @plsc.parallel_loop(0, CHUNK // L, unroll=2)
                def _fill(grp):
                    for eo in range(L):
                        sout[b, grp * L + eo, :] = neg

            pltpu.async_copy(
                sout.at[b],
                p16_hbm.at[pl.ds((wid * CPW + i) * CHUNK, CHUNK)],
                wsem)
        return carry

    lax.fori_loop(0, CPW // 2, pair_body, 0)

    for b in range(2):
        i = CPW - 2 + b
        pltpu.make_async_copy(
            sout.at[b],
            p16_hbm.at[pl.ds((wid * CPW + i) * CHUNK, CHUNK)],
            wsem).wait()


@functools.partial(
    pl.kernel,
    out_type=jax.ShapeDtypeStruct((E_PAD, L), jnp.float32),
    mesh=_MESH,
    scratch_types=[
        pltpu.VMEM((2, CHUNK), jnp.int32),
        pltpu.VMEM((2, CHUNK), jnp.int32),
        pltpu.VMEM((2, CHUNK, D), jnp.float32),
        pltpu.VMEM((2, CHUNK, D), jnp.float32),
        pltpu.VMEM((2, CHUNK, L), jnp.float32),
        pltpu.SemaphoreType.DMA,
        pltpu.SemaphoreType.DMA,
        pltpu.SemaphoreType.DMA,
    ],
)
def _scores_kernel(q_hbm, k_hbm, row_hbm, col_hbm, p16_hbm,
                   idxr, idxc, qr, kc, sout, isem, gsem, wsem):
    _scores_body(q_hbm, k_hbm, row_hbm, col_hbm, p16_hbm,
                 idxr, idxc, qr, kc, sout, isem, gsem, wsem)


def _agg_body(v_hbm, row_hbm, col_hbm, alpha_hbm, zeros_hbm, out_hbm,
              idxr, idxc, sidx, av, vrows, acc, isem, gsem, ssem):
    core = lax.axis_index("c")
    sub = lax.axis_index("s")
    wid = core * NS + sub

    def zero_body(i, carry):
        c = i * NS + sub

        @pl.when(c < NRC)
        def _():
            pltpu.sync_copy(zeros_hbm, acc.at[pl.ds(c * ROWCH, ROWCH)])

        return carry

    lax.fori_loop(0, RC_PER_SUB, zero_body, 0)
    plsc.subcore_barrier()

    def issue_idx(slot, i):
        @pl.when(i < HPW)
        def _():
            pltpu.async_copy(row_hbm.at[wid * HPW + i], idxr.at[slot], isem)
            pltpu.async_copy(col_hbm.at[wid * HPW + i], idxc.at[slot], isem)
            pltpu.async_copy(alpha_hbm.at[wid * HPW + i], av.at[slot], isem)

    def wait_idx(slot, i):
        @pl.when(i < HPW)
        def _():
            pltpu.make_async_copy(row_hbm.at[wid * HPW + i], idxr.at[slot], isem).wait()
            pltpu.make_async_copy(col_hbm.at[wid * HPW + i], idxc.at[slot], isem).wait()
            pltpu.make_async_copy(alpha_hbm.at[wid * HPW + i], av.at[slot], isem).wait()

    def issue(slot):
        pltpu.async_copy(v_hbm.at[idxc.at[slot]], vrows.at[slot], gsem)

    def wait_gathers(slot):
        pltpu.make_async_copy(v_hbm.at[idxc.at[slot]], vrows.at[slot], gsem).wait()

    def wait_scatter(slot):
        pltpu.make_async_copy(vrows.at[slot], acc.at[sidx.at[slot]],
                              ssem.at[slot]).wait()

    pltpu.sync_copy(row_hbm.at[wid * HPW], idxr.at[0])
    pltpu.sync_copy(col_hbm.at[wid * HPW], idxc.at[0])
    pltpu.sync_copy(alpha_hbm.at[wid * HPW], av.at[0])
    issue(0)
    issue_idx(1, 1)

    def quad_body(p, carry):
        for b in range(4):
            i = p * 4 + b

            wait_gathers(b)

            nslot = (b + 1) % 4

            @pl.when(i + 1 < HPW)
            def _():
                @pl.when(i >= 3)
                def _():
                    wait_scatter(nslot)
                wait_idx(nslot, i + 1)
                issue(nslot)

            issue_idx((b + 2) % 4, i + 2)

            @plsc.parallel_loop(0, CA // L, unroll=2)
            def _scale(grp):
                ag = av[b, pl.ds(grp * L, L)]
                for j in range(L):
                    e = grp * L + j
                    a = ag[j]
                    for d in range(D // L):
                        vrows[b, e, pl.ds(d * L, L)] = (
                            vrows[b, e, pl.ds(d * L, L)] * a)

            @plsc.parallel_loop(0, CA // L, unroll=2)
            def _cpidx(grp):
                sidx[b, pl.ds(grp * L, L)] = idxr[b, pl.ds(grp * L, L)]

            pltpu.async_copy(vrows.at[b], acc.at[sidx.at[b]], ssem.at[b],
                             add=True)
        return carry

    lax.fori_loop(0, HPW // 4, quad_body, 0)

    for b in range(4):
        wait_scatter((HPW - 4 + b) % 4)

    plsc.subcore_barrier()

    def out_body(i, carry):
        c = i * NS + sub

        @pl.when(c < NRC)
        def _():
            pltpu.sync_copy(
                acc.at[pl.ds(c * ROWCH, ROWCH)],
                out_hbm.at[core, pl.ds(c * ROWCH, ROWCH)],
            )

        return carry

    lax.fori_loop(0, RC_PER_SUB, out_body, 0)


@functools.partial(
    pl.kernel,
    out_type=jax.ShapeDtypeStruct((NC, N, D), jnp.float32),
    mesh=_MESH,
    scratch_types=[
        pltpu.VMEM((4, CA), jnp.int32),
        pltpu.VMEM((4, CA), jnp.int32),
        pltpu.VMEM((4, CA), jnp.int32),
        pltpu.VMEM((4, CA), jnp.float32),
        pltpu.VMEM((4, CA, D), jnp.float32),
        pltpu.VMEM_SHARED((N, D), jnp.float32),
        pltpu.SemaphoreType.DMA,
        pltpu.SemaphoreType.DMA,
        pltpu.SemaphoreType.DMA((4,)),
    ],
)
def _agg_kernel(v_hbm, row_hbm, col_hbm, alpha_hbm, zeros_hbm, out_hbm,
                idxr, idxc, sidx, av, vrows, acc, isem, gsem, ssem):
    _agg_body(v_hbm, row_hbm, col_hbm, alpha_hbm, zeros_hbm, out_hbm,
              idxr, idxc, sidx, av, vrows, acc, isem, gsem, ssem)


def _permute_edges(a):
    ap = jnp.concatenate([a, jnp.zeros((E_PAD - E,), a.dtype)])
    return ap.reshape(CPW, NW, CHUNK).transpose(1, 0, 2).reshape(
        NW * CPW, CHUNK)


def _attention_layer_sc(qkv, row2d, col2d, rmat, zeros_sub):
    q, k, v = qkv
    p16 = _scores_kernel(q, k, row2d, col2d)
    alpha2d = _softmax_call(p16, rmat)
    row64 = row2d.reshape(NW * HPW, CA)
    col64 = col2d.reshape(NW * HPW, CA)
    alpha64 = alpha2d.reshape(NW * HPW, CA)
    parts = _agg_kernel(v, row64, col64, alpha64, zeros_sub)
    return parts[0], parts[1]


@jax.jit
def kernel(x, edge_index, Wq1, bq1, Wk1, bk1, Wv1, bv1,
           Wq2, bq2, Wk2, bk2, Wv2, bv2):
    row2d = _permute_edges(edge_index[0].astype(jnp.int32))
    col2d = _permute_edges(edge_index[1].astype(jnp.int32))

    w1 = jnp.concatenate([Wq1, Wk1, Wv1], axis=1)
    b1 = jnp.concatenate([bq1, bk1, bv1]).reshape(1, 3 * D)
    w2 = jnp.concatenate([Wq2, Wk2, Wv2], axis=1)
    b2 = jnp.concatenate([bq2, bk2, bv2]).reshape(1, 3 * D)

    rmat = jnp.kron(jnp.eye(CHUNK, dtype=jnp.float32),
                    jnp.full((L, 1), 1.0 / math.sqrt(D), jnp.float32))
    zeros_sub = jnp.zeros((ROWCH, D), jnp.float32)

    qkv1 = _qkv_call(x, w1, b1)
    p0, p1 = _attention_layer_sc(qkv1, row2d, col2d, rmat, zeros_sub)
    qkv2 = _elu_qkv_call(p0, p1, w2, b2)
    p0b, p1b = _attention_layer_sc(qkv2, row2d, col2d, rmat, zeros_sub)
    return _add_call(p0b, p1b)


# speedup vs baseline: 1.0612x; 1.0612x over previous
import functools
import math

import jax
import jax.numpy as jnp
from jax import lax
from jax.experimental import pallas as pl
from jax.experimental.pallas import tpu as pltpu
from jax.experimental.pallas import tpu_sc as plsc

N = 10000
E = 320000
D = 128
L = 16
CHUNK = 128
SCH = 64
NC = 2
NS = 16
NW = NC * NS
NUM_CHUNKS = E // CHUNK
CPW = -(-NUM_CHUNKS // NW)
CPW = CPW + (CPW % 2)
E_PAD = NW * CPW * CHUNK
SNUM_CHUNKS = E // SCH
SCPW = E_PAD // (NW * SCH)
ROWCH = 200
NRC = N // ROWCH
RC_PER_SUB = -(-NRC // NS)
SM_ROWS = E_PAD * L // 128


def _qkv_body(x_ref, w_ref, b_ref, q_ref, k_ref, v_ref):
    y = jnp.dot(x_ref[...], w_ref[...], preferred_element_type=jnp.float32)
    y = y + b_ref[...]
    q_ref[...] = y[:, 0:D]
    k_ref[...] = y[:, D:2 * D]
    v_ref[...] = y[:, 2 * D:3 * D]


def _qkv_call(x, wcat, bcat):
    blk = 1000
    return pl.pallas_call(
        _qkv_body,
        grid=(N // blk,),
        in_specs=[
            pl.BlockSpec((blk, D), lambda i: (i, 0)),
            pl.BlockSpec((D, 3 * D), lambda i: (0, 0)),
            pl.BlockSpec((1, 3 * D), lambda i: (0, 0)),
        ],
        out_specs=[pl.BlockSpec((blk, D), lambda i: (i, 0))] * 3,
        out_shape=[jax.ShapeDtypeStruct((N, D), jnp.float32)] * 3,
    )(x, wcat, bcat)


def _elu_qkv_body(p0_ref, p1_ref, w_ref, b_ref, q_ref, k_ref, v_ref):
    h = p0_ref[...] + p1_ref[...]
    h = jnp.where(h > 0, h, jnp.exp(jnp.minimum(h, 0.0)) - 1.0)
    y = jnp.dot(h, w_ref[...], preferred_element_type=jnp.float32)
    y = y + b_ref[...]
    q_ref[...] = y[:, 0:D]
    k_ref[...] = y[:, D:2 * D]
    v_ref[...] = y[:, 2 * D:3 * D]


def _elu_qkv_call(p0, p1, wcat, bcat):
    blk = 1000
    return pl.pallas_call(
        _elu_qkv_body,
        grid=(N // blk,),
        in_specs=[
            pl.BlockSpec((blk, D), lambda i: (i, 0)),
            pl.BlockSpec((blk, D), lambda i: (i, 0)),
            pl.BlockSpec((D, 3 * D), lambda i: (0, 0)),
            pl.BlockSpec((1, 3 * D), lambda i: (0, 0)),
        ],
        out_specs=[pl.BlockSpec((blk, D), lambda i: (i, 0))] * 3,
        out_shape=[jax.ShapeDtypeStruct((N, D), jnp.float32)] * 3,
    )(p0, p1, wcat, bcat)


def _softmax_body(p_ref, rmat_ref, a_ref):
    s = jnp.dot(p_ref[...], rmat_ref[...], preferred_element_type=jnp.float32)
    m = jnp.max(s)
    ex = jnp.exp(s - m)
    a_ref[...] = ex * (1.0 / jnp.sum(ex))


def _softmax_call(p16, rmat):
    p2 = p16.reshape(NW * SCPW, SCH * L)
    return pl.pallas_call(
        _softmax_body,
        in_specs=[
            pl.BlockSpec((NW * SCPW, SCH * L), lambda: (0, 0)),
            pl.BlockSpec((SCH * L, SCH), lambda: (0, 0)),
        ],
        out_specs=pl.BlockSpec((NW * SCPW, SCH), lambda: (0, 0)),
        out_shape=jax.ShapeDtypeStruct((NW * SCPW, SCH), jnp.float32),
    )(p2, rmat)


def _add_body(p0_ref, p1_ref, o_ref):
    o_ref[...] = p0_ref[...] + p1_ref[...]


def _add_call(p0, p1):
    blk = 1000
    return pl.pallas_call(
        _add_body,
        grid=(N // blk,),
        in_specs=[pl.BlockSpec((blk, D), lambda i: (i, 0))] * 2,
        out_specs=pl.BlockSpec((blk, D), lambda i: (i, 0)),
        out_shape=jax.ShapeDtypeStruct((N, D), jnp.float32),
    )(p0, p1)


_MESH = plsc.VectorSubcoreMesh(core_axis_name="c", subcore_axis_name="s")


def _scores_body(q_hbm, k_hbm, row_hbm, col_hbm, p16_hbm,
                 idxr, idxc, qr, kc, sout, isem, gsem, wsem):
    core = lax.axis_index("c")
    sub = lax.axis_index("s")
    wid = core * NS + sub

    def valid(i):
        return i * NW + wid < SNUM_CHUNKS

    def issue_idx(slot, i):
        @pl.when(i < SCPW)
        def _():
            pltpu.async_copy(row_hbm.at[wid * SCPW + i], idxr.at[slot], isem)
            pltpu.async_copy(col_hbm.at[wid * SCPW + i], idxc.at[slot], isem)

    def wait_idx(slot, i):
        @pl.when(i < SCPW)
        def _():
            pltpu.make_async_copy(row_hbm.at[wid * SCPW + i], idxr.at[slot], isem).wait()
            pltpu.make_async_copy(col_hbm.at[wid * SCPW + i], idxc.at[slot], isem).wait()

    def issue(slot, i):
        @pl.when(valid(i))
        def _():
            pltpu.async_copy(q_hbm.at[idxr.at[slot]], qr.at[slot], gsem)
            pltpu.async_copy(k_hbm.at[idxc.at[slot]], kc.at[slot], gsem)

    def wait_gathers(slot, i):
        @pl.when(valid(i))
        def _():
            pltpu.make_async_copy(q_hbm.at[idxr.at[slot]], qr.at[slot], gsem).wait()
            pltpu.make_async_copy(k_hbm.at[idxc.at[slot]], kc.at[slot], gsem).wait()

    pltpu.sync_copy(row_hbm.at[wid * SCPW], idxr.at[0])
    pltpu.sync_copy(col_hbm.at[wid * SCPW], idxc.at[0])
    pltpu.sync_copy(row_hbm.at[wid * SCPW + 1], idxr.at[1])
    pltpu.sync_copy(col_hbm.at[wid * SCPW + 1], idxc.at[1])
    issue(0, 0)
    issue(1, 1)
    issue_idx(2, 2)

    def quad_body(p, carry):
        for b in range(4):
            i = p * 4 + b
            wait_gathers(b, i)
            wait_idx((b + 2) % 4, i + 2)
            issue((b + 2) % 4, i + 2)
            issue_idx((b + 3) % 4, i + 3)

            @pl.when(i >= 4)
            def _():
                pltpu.make_async_copy(
                    sout.at[b],
                    p16_hbm.at[pl.ds((wid * SCPW + i - 4) * SCH, SCH)],
                    wsem).wait()

            @pl.when(valid(i))
            def _():
                @plsc.parallel_loop(0, SCH // L, unroll=2)
                def _compute(grp):
                    for eo in range(L):
                        e = grp * L + eo
                        acc = qr[b, e, pl.ds(0, L)] * kc[b, e, pl.ds(0, L)]
                        for d in range(1, D // L):
                            acc = acc + (qr[b, e, pl.ds(d * L, L)] *
                                         kc[b, e, pl.ds(d * L, L)])
                        sout[b, e, :] = acc

            @pl.when(jnp.logical_not(valid(i)))
            def _():
                neg = jnp.full((L,), -1.0e30, jnp.float32)

                @plsc.parallel_loop(0, SCH // L, unroll=2)
                def _fill(grp):
                    for eo in range(L):
                        sout[b, grp * L + eo, :] = neg

            pltpu.async_copy(
                sout.at[b],
                p16_hbm.at[pl.ds((wid * SCPW + i) * SCH, SCH)],
                wsem)
        return carry

    lax.fori_loop(0, SCPW // 4, quad_body, 0)

    for b in range(4):
        i = SCPW - 4 + b
        pltpu.make_async_copy(
            sout.at[b],
            p16_hbm.at[pl.ds((wid * SCPW + i) * SCH, SCH)],
            wsem).wait()


@functools.partial(
    pl.kernel,
    out_type=jax.ShapeDtypeStruct((E_PAD, L), jnp.float32),
    mesh=_MESH,
    scratch_types=[
        pltpu.VMEM((4, SCH), jnp.int32),
        pltpu.VMEM((4, SCH), jnp.int32),
        pltpu.VMEM((4, SCH, D), jnp.float32),
        pltpu.VMEM((4, SCH, D), jnp.float32),
        pltpu.VMEM((4, SCH, L), jnp.float32),
        pltpu.SemaphoreType.DMA,
        pltpu.SemaphoreType.DMA,
        pltpu.SemaphoreType.DMA,
    ],
)
def _scores_kernel(q_hbm, k_hbm, row_hbm, col_hbm, p16_hbm,
                   idxr, idxc, qr, kc, sout, isem, gsem, wsem):
    _scores_body(q_hbm, k_hbm, row_hbm, col_hbm, p16_hbm,
                 idxr, idxc, qr, kc, sout, isem, gsem, wsem)


def _agg_body(v_hbm, row_hbm, col_hbm, alpha_hbm, zeros_hbm, out_hbm,
              idxr, idxc, sidx, av, vrows, acc, isem, gsem, ssem):
    core = lax.axis_index("c")
    sub = lax.axis_index("s")
    wid = core * NS + sub

    def zero_body(i, carry):
        c = i * NS + sub

        @pl.when(c < NRC)
        def _():
            pltpu.sync_copy(zeros_hbm, acc.at[pl.ds(c * ROWCH, ROWCH)])

        return carry

    lax.fori_loop(0, RC_PER_SUB, zero_body, 0)
    plsc.subcore_barrier()

    def issue_idx(slot, i):
        @pl.when(i < CPW)
        def _():
            pltpu.async_copy(row_hbm.at[wid * CPW + i], idxr.at[slot], isem)
            pltpu.async_copy(col_hbm.at[wid * CPW + i], idxc.at[slot], isem)
            pltpu.async_copy(alpha_hbm.at[wid * CPW + i], av.at[slot], isem)

    def wait_idx(slot, i):
        @pl.when(i < CPW)
        def _():
            pltpu.make_async_copy(row_hbm.at[wid * CPW + i], idxr.at[slot], isem).wait()
            pltpu.make_async_copy(col_hbm.at[wid * CPW + i], idxc.at[slot], isem).wait()
            pltpu.make_async_copy(alpha_hbm.at[wid * CPW + i], av.at[slot], isem).wait()

    def issue(slot):
        pltpu.async_copy(v_hbm.at[idxc.at[slot]], vrows.at[slot], gsem)

    def wait_gathers(slot):
        pltpu.make_async_copy(v_hbm.at[idxc.at[slot]], vrows.at[slot], gsem).wait()

    def wait_scatter(slot):
        pltpu.make_async_copy(vrows.at[slot], acc.at[sidx.at[slot]], ssem).wait()

    pltpu.sync_copy(row_hbm.at[wid * CPW], idxr.at[0])
    pltpu.sync_copy(col_hbm.at[wid * CPW], idxc.at[0])
    pltpu.sync_copy(alpha_hbm.at[wid * CPW], av.at[0])
    issue(0)
    issue_idx(1, 1)

    def pair_body(p, carry):
        for b in range(2):
            i = p * 2 + b

            wait_gathers(b)

            @pl.when(i >= 1)
            def _():
                wait_scatter(1 - b)

            @pl.when(i + 1 < CPW)
            def _():
                wait_idx(1 - b, i + 1)
                issue(1 - b)

            @plsc.parallel_loop(0, CHUNK // L, unroll=2)
            def _scale(grp):
                ag = av[b, pl.ds(grp * L, L)]
                for j in range(L):
                    e = grp * L + j
                    a = ag[j]
                    for d in range(D // L):
                        vrows[b, e, pl.ds(d * L, L)] = (
                            vrows[b, e, pl.ds(d * L, L)] * a)

            @plsc.parallel_loop(0, CHUNK // L, unroll=2)
            def _cpidx(grp):
                sidx[b, pl.ds(grp * L, L)] = idxr[b, pl.ds(grp * L, L)]

            pltpu.async_copy(vrows.at[b], acc.at[sidx.at[b]], ssem, add=True)

            issue_idx(b, i + 2)
        return carry

    lax.fori_loop(0, CPW // 2, pair_body, 0)

    wait_scatter((CPW - 1) % 2)

    plsc.subcore_barrier()

    def out_body(i, carry):
        c = i * NS + sub

        @pl.when(c < NRC)
        def _():
            pltpu.sync_copy(
                acc.at[pl.ds(c * ROWCH, ROWCH)],
                out_hbm.at[core, pl.ds(c * ROWCH, ROWCH)],
            )

        return carry

    lax.fori_loop(0, RC_PER_SUB, out_body, 0)


@functools.partial(
    pl.kernel,
    out_type=jax.ShapeDtypeStruct((NC, N, D), jnp.float32),
    mesh=_MESH,
    scratch_types=[
        pltpu.VMEM((2, CHUNK), jnp.int32),
        pltpu.VMEM((2, CHUNK), jnp.int32),
        pltpu.VMEM((2, CHUNK), jnp.int32),
        pltpu.VMEM((2, CHUNK), jnp.float32),
        pltpu.VMEM((2, CHUNK, D), jnp.float32),
        pltpu.VMEM_SHARED((N, D), jnp.float32),
        pltpu.SemaphoreType.DMA,
        pltpu.SemaphoreType.DMA,
        pltpu.SemaphoreType.DMA,
    ],
)
def _agg_kernel(v_hbm, row_hbm, col_hbm, alpha_hbm, zeros_hbm, out_hbm,
                idxr, idxc, sidx, av, vrows, acc, isem, gsem, ssem):
    _agg_body(v_hbm, row_hbm, col_hbm, alpha_hbm, zeros_hbm, out_hbm,
              idxr, idxc, sidx, av, vrows, acc, isem, gsem, ssem)


def _permute_edges(a):
    ap = jnp.concatenate([a, jnp.zeros((E_PAD - E,), a.dtype)])
    return ap.reshape(SCPW, NW, SCH).transpose(1, 0, 2).reshape(
        NW * SCPW, SCH)


def _attention_layer_sc(qkv, rowS, colS, rowA, colA, rmat, zeros_sub):
    q, k, v = qkv
    p16 = _scores_kernel(q, k, rowS, colS)
    alpha2d = _softmax_call(p16, rmat)
    alphaA = alpha2d.reshape(NW * CPW, CHUNK)
    parts = _agg_kernel(v, rowA, colA, alphaA, zeros_sub)
    return parts[0], parts[1]


@jax.jit
def kernel(x, edge_index, Wq1, bq1, Wk1, bk1, Wv1, bv1,
           Wq2, bq2, Wk2, bk2, Wv2, bv2):
    rowS = _permute_edges(edge_index[0].astype(jnp.int32))
    colS = _permute_edges(edge_index[1].astype(jnp.int32))
    rowA = rowS.reshape(NW * CPW, CHUNK)
    colA = colS.reshape(NW * CPW, CHUNK)

    w1 = jnp.concatenate([Wq1, Wk1, Wv1], axis=1)
    b1 = jnp.concatenate([bq1, bk1, bv1]).reshape(1, 3 * D)
    w2 = jnp.concatenate([Wq2, Wk2, Wv2], axis=1)
    b2 = jnp.concatenate([bq2, bk2, bv2]).reshape(1, 3 * D)

    rmat = jnp.kron(jnp.eye(SCH, dtype=jnp.float32),
                    jnp.full((L, 1), 1.0 / math.sqrt(D), jnp.float32))
    zeros_sub = jnp.zeros((ROWCH, D), jnp.float32)

    qkv1 = _qkv_call(x, w1, b1)
    p0, p1 = _attention_layer_sc(qkv1, rowS, colS, rowA, colA, rmat,
                                 zeros_sub)
    qkv2 = _elu_qkv_call(p0, p1, w2, b2)
    p0b, p1b = _attention_layer_sc(qkv2, rowS, colS, rowA, colA, rmat,
                                   zeros_sub)
    return _add_call(p0b, p1b)
